# Initial kernel scaffold; baseline (speedup 1.0000x reference)
import jax
import jax.numpy as jnp
from jax.experimental import pallas as pl


def kernel(x, qr, cos, sin, Wq_b, Wk, kn_w, kn_b, Wproj):
    raise NotImplementedError("write your pallas kernel here")


# speedup vs baseline: 1.1838x; 1.1838x over previous
import jax
import jax.numpy as jnp
from jax.experimental import pallas as pl

B, S, HID, LR = 1, 2048, 2048, 1536
H, D, RD, TOPK = 8, 64, 64, 256
BQ = 256

_HIGH = jax.lax.Precision.HIGHEST


def _rope(v, c, s):
    half = v.shape[-1] // 2
    rot = jnp.concatenate([-v[..., half:], v[..., :half]], axis=-1)
    return v * c + rot * s


def _qk_kernel(x_ref, qr_ref, cos_ref, sin_ref, wqb_ref, wk_ref, knw_ref,
               knb_ref, wproj_ref, q_ref, k_ref, gate_ref):
    x = x_ref[...].astype(jnp.bfloat16)
    c = cos_ref[...]
    s = sin_ref[...]
    k = jax.lax.dot_general(x, wk_ref[...].astype(jnp.bfloat16),
                            (((1,), (1,)), ((), ())),
                            preferred_element_type=jnp.float32)
    mu = jnp.mean(k, axis=-1, keepdims=True)
    var = jnp.mean((k - mu) ** 2, axis=-1, keepdims=True)
    k = (k - mu) * jax.lax.rsqrt(var + 1e-5) * knw_ref[...] + knb_ref[...]
    k_ref[...] = _rope(k, c, s).astype(jnp.bfloat16)
    gate_ref[...] = jax.lax.dot_general(
        x, wproj_ref[...].astype(jnp.bfloat16), (((1,), (1,)), ((), ())),
        preferred_element_type=jnp.float32).astype(jnp.bfloat16)
    qfull = jax.lax.dot_general(qr_ref[...].astype(jnp.bfloat16),
                                wqb_ref[...].astype(jnp.bfloat16),
                                (((1,), (1,)), ((), ())),
                                preferred_element_type=jnp.float32)
    for h in range(H):
        q_ref[:, h * D:(h + 1) * D] = (
            _rope(qfull[:, h * D:(h + 1) * D], c, s).astype(jnp.bfloat16))


def _score_kernel(q_ref, k_ref, gate_ref, out_ref):
    scale = D ** -0.5
    gate = gate_ref[...].astype(jnp.float32)
    acc = jnp.zeros((BQ, S), jnp.float32)
    for h in range(H):
        sh = jax.lax.dot_general(q_ref[:, h * D:(h + 1) * D], k_ref[...],
                                 (((1,), (1,)), ((), ())),
                                 preferred_element_type=jnp.float32)
        sh = (sh * scale).astype(jnp.bfloat16).astype(jnp.float32)
        acc = acc + gate[:, h:h + 1] * sh
    out_ref[...] = acc


def kernel(x, qr, cos, sin, Wq_b, Wk, kn_w, kn_b, Wproj):
    x2 = x[0]
    qr2 = qr[0]
    cos2 = cos[0]
    sin2 = sin[0]
    nblk = S // BQ
    q, k, gate = pl.pallas_call(
        _qk_kernel,
        grid=(nblk,),
        in_specs=[
            pl.BlockSpec((BQ, HID), lambda i: (i, 0)),
            pl.BlockSpec((BQ, LR), lambda i: (i, 0)),
            pl.BlockSpec((BQ, RD), lambda i: (i, 0)),
            pl.BlockSpec((BQ, RD), lambda i: (i, 0)),
            pl.BlockSpec((H * D, LR), lambda i: (0, 0)),
            pl.BlockSpec((D, HID), lambda i: (0, 0)),
            pl.BlockSpec((D,), lambda i: (0,)),
            pl.BlockSpec((D,), lambda i: (0,)),
            pl.BlockSpec((H, HID), lambda i: (0, 0)),
        ],
        out_specs=[
            pl.BlockSpec((BQ, H * D), lambda i: (i, 0)),
            pl.BlockSpec((BQ, D), lambda i: (i, 0)),
            pl.BlockSpec((BQ, H), lambda i: (i, 0)),
        ],
        out_shape=[
            jax.ShapeDtypeStruct((S, H * D), jnp.bfloat16),
            jax.ShapeDtypeStruct((S, D), jnp.bfloat16),
            jax.ShapeDtypeStruct((S, H), jnp.bfloat16),
        ],
    )(x2, qr2, cos2, sin2, Wq_b, Wk, kn_w, kn_b, Wproj)

    scores = pl.pallas_call(
        _score_kernel,
        grid=(nblk,),
        in_specs=[
            pl.BlockSpec((BQ, H * D), lambda i: (i, 0)),
            pl.BlockSpec((S, D), lambda i: (0, 0)),
            pl.BlockSpec((BQ, H), lambda i: (i, 0)),
        ],
        out_specs=pl.BlockSpec((BQ, S), lambda i: (i, 0)),
        out_shape=jax.ShapeDtypeStruct((S, S), jnp.float32),
    )(q, k, gate)

    _, idx = jax.lax.top_k(scores, TOPK)
    return idx[None]


# speedup vs baseline: 5.1055x; 4.3128x over previous
import functools

import jax
import jax.numpy as jnp
from jax import lax
from jax.experimental import pallas as pl
from jax.experimental.pallas import tpu as pltpu
from jax.experimental.pallas import tpu_sc as plsc

B, S, HID, LR = 1, 2048, 2048, 1536
H, D, RD, TOPK = 8, 64, 64, 256
BQ = 256
NW = 32
VL = 16
CHUNK = 8

_HIGH = jax.lax.Precision.HIGHEST
_INF = 0x7FFFFFFF


def _rope(v, c, s):
    half = v.shape[-1] // 2
    rot = jnp.concatenate([-v[..., half:], v[..., :half]], axis=-1)
    return v * c + rot * s


def _qk_kernel(x_ref, qr_ref, cos_ref, sin_ref, wqb_ref, wk_ref, knw_ref,
               knb_ref, wproj_ref, q_ref, k_ref, gate_ref):
    x = x_ref[...].astype(jnp.bfloat16)
    c = cos_ref[...]
    s = sin_ref[...]
    k = jax.lax.dot_general(x, wk_ref[...].astype(jnp.bfloat16),
                            (((1,), (1,)), ((), ())),
                            preferred_element_type=jnp.float32)
    mu = jnp.mean(k, axis=-1, keepdims=True)
    var = jnp.mean((k - mu) ** 2, axis=-1, keepdims=True)
    k = (k - mu) * jax.lax.rsqrt(var + 1e-5) * knw_ref[...] + knb_ref[...]
    k_ref[...] = _rope(k, c, s).astype(jnp.bfloat16)
    gate_ref[...] = jax.lax.dot_general(
        x, wproj_ref[...].astype(jnp.bfloat16), (((1,), (1,)), ((), ())),
        preferred_element_type=jnp.float32).astype(jnp.bfloat16)
    qfull = jax.lax.dot_general(qr_ref[...].astype(jnp.bfloat16),
                                wqb_ref[...].astype(jnp.bfloat16),
                                (((1,), (1,)), ((), ())),
                                preferred_element_type=jnp.float32)
    for h in range(H):
        q_ref[:, h * D:(h + 1) * D] = (
            _rope(qfull[:, h * D:(h + 1) * D], c, s).astype(jnp.bfloat16))


def _score_kernel(q_ref, k_ref, gate_ref, out_ref):
    scale = D ** -0.5
    gate = gate_ref[...].astype(jnp.float32)
    acc = jnp.zeros((BQ, S), jnp.float32)
    for h in range(H):
        sh = jax.lax.dot_general(q_ref[:, h * D:(h + 1) * D], k_ref[...],
                                 (((1,), (1,)), ((), ())),
                                 preferred_element_type=jnp.float32)
        sh = (sh * scale).astype(jnp.bfloat16).astype(jnp.float32)
        acc = acc + gate[:, h:h + 1] * sh
    u = jax.lax.bitcast_convert_type(acc, jnp.int32)
    out_ref[...] = jnp.where(u >= 0, ~u, u ^ jnp.int32(-2147483648))


def _cmpex(ka, va, kb, vb):
    m = ka <= kb
    return ((jnp.minimum(ka, kb), jnp.where(m, va, vb)),
            (jnp.maximum(ka, kb), jnp.where(m, vb, va)))


def _bitonic_finish(lst):
    lst = list(lst)
    n = len(lst)
    d = n // 2
    while d >= 1:
        for g in range(0, n, 2 * d):
            for i in range(g, g + d):
                lo, hi = _cmpex(*lst[i], *lst[i + d])
                lst[i], lst[i + d] = lo, hi
        d //= 2
    return [plsc.sort_key_val(k, v) for (k, v) in lst]


def _rev_pair(kv):
    return (lax.rev(kv[0], (0,)), lax.rev(kv[1], (0,)))


def _merge_asc(a, b):
    return _bitonic_finish(a + [_rev_pair(kv) for kv in reversed(b)])


def _sort_block(pairs):
    runs = [[plsc.sort_key_val(k, v)] for (k, v) in pairs]
    while len(runs) > 1:
        nxt = [_merge_asc(runs[i], runs[i + 1])
               for i in range(0, len(runs) - 1, 2)]
        if len(runs) % 2:
            nxt.append(runs[-1])
        runs = nxt
    return runs[0]


def _merge_prune(t, bs):
    lo = []
    for j in range(len(t)):
        bk, bv = _rev_pair(bs[len(t) - 1 - j])
        tk, tv = t[j]
        m = tk <= bk
        lo.append((jnp.minimum(tk, bk), jnp.where(m, tv, bv)))
    return _bitonic_finish(lo)


def _wid():
    return lax.axis_index("s") * 2 + lax.axis_index("c")


@functools.cache
def _make_topk_cached(nrows, ncols, kk, interpret=False):
    rpw = nrows // NW
    nchunk = rpw // CHUNK
    nblk = ncols // kk
    kv = kk // VL
    mesh = plsc.VectorSubcoreMesh(core_axis_name="c", subcore_axis_name="s")

    @functools.partial(
        pl.kernel,
        out_type=jax.ShapeDtypeStruct((nrows, kk), jnp.int32),
        mesh=mesh,
        scratch_types=[pltpu.VMEM((CHUNK, ncols), jnp.int32),
                       pltpu.VMEM((CHUNK, kk), jnp.int32)],
        compiler_params=pltpu.CompilerParams(needs_layout_passes=False),
        interpret=interpret,
    )
    def topk(keys_hbm, out_hbm, inbuf, outbuf):
        wid = _wid()
        base = wid * rpw

        def chunk_body(ci, carry):
            r0 = base + ci * CHUNK
            pltpu.sync_copy(keys_hbm.at[pl.ds(r0, CHUNK), :], inbuf)

            def row_body(rr, c2):
                def blk_body(b, tflat):
                    t = [(tflat[2 * j], tflat[2 * j + 1]) for j in range(kv)]
                    pairs = []
                    for j in range(kv):
                        k = inbuf[rr, pl.ds(b * kk + j * VL, VL)]
                        v = lax.iota(jnp.int32, VL) + (b * kk + j * VL)
                        pairs.append((k, v))
                    t = _merge_prune(t, _sort_block(pairs))
                    return tuple(x for kvp in t for x in kvp)

                init = []
                for _ in range(kv):
                    init += [jnp.full((VL,), _INF, jnp.int32),
                         jnp.zeros((VL,), jnp.int32)]
                tflat = lax.fori_loop(0, nblk, blk_body, tuple(init))
                for j in range(kv):
                    outbuf[rr, pl.ds(j * VL, VL)] = tflat[2 * j + 1]
                return c2

            lax.fori_loop(0, CHUNK, row_body, 0)
            pltpu.sync_copy(outbuf, out_hbm.at[pl.ds(r0, CHUNK), :])
            return carry

        lax.fori_loop(0, nchunk, chunk_body, 0)

    return topk


def kernel(x, qr, cos, sin, Wq_b, Wk, kn_w, kn_b, Wproj):
    x2 = x[0]
    qr2 = qr[0]
    cos2 = cos[0]
    sin2 = sin[0]
    nblk = S // BQ
    q, k, gate = pl.pallas_call(
        _qk_kernel,
        grid=(nblk,),
        in_specs=[
            pl.BlockSpec((BQ, HID), lambda i: (i, 0)),
            pl.BlockSpec((BQ, LR), lambda i: (i, 0)),
            pl.BlockSpec((BQ, RD), lambda i: (i, 0)),
            pl.BlockSpec((BQ, RD), lambda i: (i, 0)),
            pl.BlockSpec((H * D, LR), lambda i: (0, 0)),
            pl.BlockSpec((D, HID), lambda i: (0, 0)),
            pl.BlockSpec((D,), lambda i: (0,)),
            pl.BlockSpec((D,), lambda i: (0,)),
            pl.BlockSpec((H, HID), lambda i: (0, 0)),
        ],
        out_specs=[
            pl.BlockSpec((BQ, H * D), lambda i: (i, 0)),
            pl.BlockSpec((BQ, D), lambda i: (i, 0)),
            pl.BlockSpec((BQ, H), lambda i: (i, 0)),
        ],
        out_shape=[
            jax.ShapeDtypeStruct((S, H * D), jnp.bfloat16),
            jax.ShapeDtypeStruct((S, D), jnp.bfloat16),
            jax.ShapeDtypeStruct((S, H), jnp.bfloat16),
        ],
    )(x2, qr2, cos2, sin2, Wq_b, Wk, kn_w, kn_b, Wproj)

    keys = pl.pallas_call(
        _score_kernel,
        grid=(nblk,),
        in_specs=[
            pl.BlockSpec((BQ, H * D), lambda i: (i, 0)),
            pl.BlockSpec((S, D), lambda i: (0, 0)),
            pl.BlockSpec((BQ, H), lambda i: (i, 0)),
        ],
        out_specs=pl.BlockSpec((BQ, S), lambda i: (i, 0)),
        out_shape=jax.ShapeDtypeStruct((S, S), jnp.int32),
    )(q, k, gate)

    idx = _make_topk_cached(S, S, TOPK)(keys)
    return idx[None]
